# baseline (speedup 1.0000x reference)
import jax
import jax.numpy as jnp
from jax.experimental import pallas as pl
from jax.experimental.pallas import tpu as pltpu

_NCHUNK = 8


def _dma_staged_kernel(x_ref, a_ref, b_ref, bufs, in_sems, a_sems, b_sems):
    B = x_ref.shape[0]
    c = B // _NCHUNK
    ins, outs = [], []
    for i in range(_NCHUNK):
        sl = pl.ds(i * c, c)
        cp = pltpu.make_async_copy(x_ref.at[sl], bufs.at[i], in_sems.at[i])
        cp.start()
        ins.append(cp)
        outs.append((
            pltpu.make_async_copy(bufs.at[i], a_ref.at[sl], a_sems.at[i]),
            pltpu.make_async_copy(bufs.at[i], b_ref.at[sl], b_sems.at[i]),
        ))
    for i in range(_NCHUNK):
        ins[i].wait()
        outs[i][0].start()
        outs[i][1].start()
    for ca, cb in outs:
        ca.wait()
        cb.wait()


def kernel(x, mask):
    B, S, D = x.shape
    out = pl.pallas_call(
        _dma_staged_kernel,
        in_specs=[pl.BlockSpec(memory_space=pl.ANY)],
        out_specs=[
            pl.BlockSpec(memory_space=pl.ANY),
            pl.BlockSpec(memory_space=pl.ANY),
        ],
        out_shape=[
            jax.ShapeDtypeStruct(x.shape, x.dtype),
            jax.ShapeDtypeStruct(x.shape, x.dtype),
        ],
        scratch_shapes=[
            pltpu.VMEM((_NCHUNK, B // _NCHUNK, S, D), x.dtype),
            pltpu.SemaphoreType.DMA((_NCHUNK,)),
            pltpu.SemaphoreType.DMA((_NCHUNK,)),
            pltpu.SemaphoreType.DMA((_NCHUNK,)),
        ],
        compiler_params=pltpu.CompilerParams(
            vmem_limit_bytes=128 * 1024 * 1024,
        ),
    )(x)
    return (out[0], out[1])


# speedup vs baseline: 1.0131x; 1.0131x over previous
import jax
import jax.numpy as jnp
from jax.experimental import pallas as pl
from jax.experimental.pallas import tpu as pltpu

_NCHUNK = 2


def _dma_staged_kernel(x_ref, a_ref, b_ref, bufs, in_sems, a_sems, b_sems):
    B = x_ref.shape[0]
    c = B // _NCHUNK
    ins, outs = [], []
    for i in range(_NCHUNK):
        sl = pl.ds(i * c, c)
        cp = pltpu.make_async_copy(x_ref.at[sl], bufs.at[i], in_sems.at[i])
        cp.start()
        ins.append(cp)
        outs.append((
            pltpu.make_async_copy(bufs.at[i], a_ref.at[sl], a_sems.at[i]),
            pltpu.make_async_copy(bufs.at[i], b_ref.at[sl], b_sems.at[i]),
        ))
    for i in range(_NCHUNK):
        ins[i].wait()
        outs[i][0].start()
        outs[i][1].start()
    for ca, cb in outs:
        ca.wait()
        cb.wait()


def kernel(x, mask):
    B, S, D = x.shape
    out = pl.pallas_call(
        _dma_staged_kernel,
        in_specs=[pl.BlockSpec(memory_space=pl.ANY)],
        out_specs=[
            pl.BlockSpec(memory_space=pl.ANY),
            pl.BlockSpec(memory_space=pl.ANY),
        ],
        out_shape=[
            jax.ShapeDtypeStruct(x.shape, x.dtype),
            jax.ShapeDtypeStruct(x.shape, x.dtype),
        ],
        scratch_shapes=[
            pltpu.VMEM((_NCHUNK, B // _NCHUNK, S, D), x.dtype),
            pltpu.SemaphoreType.DMA((_NCHUNK,)),
            pltpu.SemaphoreType.DMA((_NCHUNK,)),
            pltpu.SemaphoreType.DMA((_NCHUNK,)),
        ],
        compiler_params=pltpu.CompilerParams(
            vmem_limit_bytes=128 * 1024 * 1024,
        ),
    )(x)
    return (out[0], out[1])
